# baseline (speedup 1.0000x reference)
import functools

import jax
import jax.numpy as jnp
from jax import lax
from jax.experimental import pallas as pl
from jax.experimental.pallas import tpu as pltpu
from jax.experimental.pallas import tpu_sc as plsc

_L = 16


def _build_sc_call(batch, height, width, num_feats):
    hw = height * width
    two_d = 2 * num_feats
    n_workers = 32
    chans_per_w = two_d // n_workers

    mesh = plsc.VectorSubcoreMesh(core_axis_name="c", subcore_axis_name="s")

    @functools.partial(
        pl.kernel,
        mesh=mesh,
        out_type=jax.ShapeDtypeStruct((batch, two_d, height, width), jnp.float32),
        scratch_types=[
            pltpu.VMEM((width * num_feats,), jnp.float32),
            pltpu.VMEM((height * num_feats,), jnp.float32),
            pltpu.VMEM((chans_per_w, height, width), jnp.float32),
            pltpu.SemaphoreType.DMA,
        ],
        compiler_params=pltpu.CompilerParams(
            needs_layout_passes=False,
            skip_device_barrier=True,
            disable_bounds_checks=True,
            disable_semaphore_checks=True,
        ),
    )
    def _k(row_hbm, col_hbm, out_hbm, colv, rowv, blk, sem):
        wid = lax.axis_index("s") * 2 + lax.axis_index("c")
        is_col = wid < (n_workers // 2)

        pltpu.sync_copy(col_hbm.at[pl.ds(0, width * num_feats)], colv)
        pltpu.sync_copy(row_hbm.at[pl.ds(0, height * num_feats)], rowv)

        iot = lax.iota(jnp.int32, _L)
        zeros = jnp.zeros((_L,), jnp.int32)

        @pl.when(is_col)
        def _():
            for c in range(chans_per_w):
                cvec = zeros + (wid * chans_per_w + c)
                v0 = plsc.load_gather(colv, [iot * num_feats + cvec])
                v1 = plsc.load_gather(colv, [(iot + _L) * num_feats + cvec])

                def body(h, _, c=c, v0=v0, v1=v1):
                    blk[c, h, pl.ds(0, _L)] = v0
                    blk[c, h, pl.ds(_L, _L)] = v1
                    return 0

                lax.fori_loop(0, height, body, 0)

        @pl.when(jnp.logical_not(is_col))
        def _():
            for c in range(chans_per_w):
                cvec = zeros + (wid * chans_per_w + c - num_feats)

                def body(h, _, c=c, cvec=cvec):
                    v = plsc.load_gather(rowv, [h * num_feats + cvec])
                    blk[c, h, pl.ds(0, _L)] = v
                    blk[c, h, pl.ds(_L, _L)] = v
                    return 0

                lax.fori_loop(0, height, body, 0)

        copies = []
        for b in range(batch):
            copies.append(
                pltpu.async_copy(
                    blk,
                    out_hbm.at[b, pl.ds(wid * chans_per_w, chans_per_w)],
                    sem,
                )
            )
        for cp in copies:
            cp.wait()

    return _k


def kernel(pixel_values, pixel_mask, row_embed, col_embed):
    batch = pixel_values.shape[0]
    height, width = pixel_values.shape[-2:]
    num_feats = row_embed.shape[1]
    call = _build_sc_call(batch, height, width, num_feats)
    return call(row_embed.reshape(-1), col_embed.reshape(-1))


# speedup vs baseline: 1.2801x; 1.2801x over previous
import jax
import jax.numpy as jnp
from jax.experimental import pallas as pl
from jax.experimental.pallas import tpu as pltpu


def _build_tc_call(batch, height, width, num_feats, table_rows):
    def body(row_ref, col_ref, o_ref):
        half = pl.program_id(1)

        @pl.when(half == 0)
        def _():
            col_t = col_ref[0:width, :].T
            o_ref[0] = jnp.broadcast_to(
                col_t[:, None, :], (num_feats, height, width)
            )

        @pl.when(half == 1)
        def _():
            row_t = row_ref[0:height, :].T
            o_ref[0] = jnp.broadcast_to(
                row_t[:, :, None], (num_feats, height, width)
            )

    return pl.pallas_call(
        body,
        grid=(batch, 2),
        in_specs=[
            pl.BlockSpec((table_rows, num_feats), lambda b, s: (0, 0)),
            pl.BlockSpec((table_rows, num_feats), lambda b, s: (0, 0)),
        ],
        out_specs=pl.BlockSpec(
            (1, num_feats, height, width), lambda b, s: (b, s, 0, 0)
        ),
        out_shape=jax.ShapeDtypeStruct(
            (batch, 2 * num_feats, height, width), jnp.float32
        ),
        compiler_params=pltpu.CompilerParams(
            dimension_semantics=("arbitrary", "arbitrary"),
        ),
    )


def kernel(pixel_values, pixel_mask, row_embed, col_embed):
    batch = pixel_values.shape[0]
    height, width = pixel_values.shape[-2:]
    table_rows, num_feats = row_embed.shape
    call = _build_tc_call(batch, height, width, num_feats, table_rows)
    return call(row_embed, col_embed)


# speedup vs baseline: 3.4469x; 2.6928x over previous
import jax
import jax.numpy as jnp
from jax import lax
from jax.experimental import pallas as pl
from jax.experimental.pallas import tpu as pltpu


def _build_tc_call(batch, height, width, num_feats, table_rows):
    hw = height * width
    two_d = 2 * num_feats

    def body(row_ref, col_ref, o_ref):
        p_col = lax.broadcasted_iota(jnp.int32, (width, hw), 1)
        w_idx = lax.broadcasted_iota(jnp.int32, (width, hw), 0)
        s_col = (p_col % width == w_idx).astype(jnp.float32)
        p_row = lax.broadcasted_iota(jnp.int32, (height, hw), 1)
        h_idx = lax.broadcasted_iota(jnp.int32, (height, hw), 0)
        s_row = (p_row // width == h_idx).astype(jnp.float32)

        col_hw = col_ref[0:width, :]
        row_hw = row_ref[0:height, :]
        x = lax.dot_general(
            col_hw, s_col, (((0,), (0,)), ((), ())),
            preferred_element_type=jnp.float32,
        )
        y = lax.dot_general(
            row_hw, s_row, (((0,), (0,)), ((), ())),
            preferred_element_type=jnp.float32,
        )
        o_ref[0, 0:num_feats, :] = x
        o_ref[0, num_feats:two_d, :] = y

    return pl.pallas_call(
        body,
        grid=(batch,),
        in_specs=[
            pl.BlockSpec((table_rows, num_feats), lambda b: (0, 0)),
            pl.BlockSpec((table_rows, num_feats), lambda b: (0, 0)),
        ],
        out_specs=pl.BlockSpec((1, two_d, hw), lambda b: (b, 0, 0)),
        out_shape=jax.ShapeDtypeStruct((batch, two_d, hw), jnp.float32),
        compiler_params=pltpu.CompilerParams(
            dimension_semantics=("arbitrary",),
        ),
    )


def kernel(pixel_values, pixel_mask, row_embed, col_embed):
    batch = pixel_values.shape[0]
    height, width = pixel_values.shape[-2:]
    table_rows, num_feats = row_embed.shape
    call = _build_tc_call(batch, height, width, num_feats, table_rows)
    out = call(row_embed, col_embed)
    return out.reshape(batch, 2 * num_feats, height, width)


# speedup vs baseline: 3.5734x; 1.0367x over previous
import jax
import jax.numpy as jnp
from jax import lax
from jax.experimental import pallas as pl
from jax.experimental.pallas import tpu as pltpu


def _build_tc_call(batch, height, width, num_feats, table_rows):
    hw = height * width
    two_d = 2 * num_feats

    def body(row_ref, col_ref, o_ref, blk):
        @pl.when(pl.program_id(0) == 0)
        def _():
            p_col = lax.broadcasted_iota(jnp.int32, (width, hw), 1)
            w_idx = lax.broadcasted_iota(jnp.int32, (width, hw), 0)
            s_col = (p_col % width == w_idx).astype(jnp.float32)
            p_row = lax.broadcasted_iota(jnp.int32, (height, hw), 1)
            h_idx = lax.broadcasted_iota(jnp.int32, (height, hw), 0)
            s_row = (p_row // width == h_idx).astype(jnp.float32)

            col_hw = col_ref[0:width, :]
            row_hw = row_ref[0:height, :]
            x = lax.dot_general(
                col_hw, s_col, (((0,), (0,)), ((), ())),
                preferred_element_type=jnp.float32,
            )
            y = lax.dot_general(
                row_hw, s_row, (((0,), (0,)), ((), ())),
                preferred_element_type=jnp.float32,
            )
            blk[0:num_feats, :] = x
            blk[num_feats:two_d, :] = y

        o_ref[0] = blk[...]

    return pl.pallas_call(
        body,
        grid=(batch,),
        in_specs=[
            pl.BlockSpec((table_rows, num_feats), lambda b: (0, 0)),
            pl.BlockSpec((table_rows, num_feats), lambda b: (0, 0)),
        ],
        out_specs=pl.BlockSpec((1, two_d, hw), lambda b: (b, 0, 0)),
        out_shape=jax.ShapeDtypeStruct((batch, two_d, hw), jnp.float32),
        scratch_shapes=[pltpu.VMEM((two_d, hw), jnp.float32)],
        compiler_params=pltpu.CompilerParams(
            dimension_semantics=("arbitrary",),
        ),
    )


def kernel(pixel_values, pixel_mask, row_embed, col_embed):
    batch = pixel_values.shape[0]
    height, width = pixel_values.shape[-2:]
    table_rows, num_feats = row_embed.shape
    call = _build_tc_call(batch, height, width, num_feats, table_rows)
    out = call(row_embed, col_embed)
    return out.reshape(batch, 2 * num_feats, height, width)


# speedup vs baseline: 3.6863x; 1.0316x over previous
import jax
import jax.numpy as jnp
from jax import lax
from jax.experimental import pallas as pl
from jax.experimental.pallas import tpu as pltpu


def _build_tc_call(batch, height, width, num_feats, table_rows):
    hw = height * width
    two_d = 2 * num_feats

    def body(row_ref, col_ref, o_ref, blk):
        @pl.when(pl.program_id(0) == 0)
        def _():
            p_col = lax.broadcasted_iota(jnp.int32, (width, hw), 1)
            w_idx = lax.broadcasted_iota(jnp.int32, (width, hw), 0)
            s_col = (p_col % width == w_idx).astype(jnp.float32)
            p_row = lax.broadcasted_iota(jnp.int32, (height, hw), 1)
            h_idx = lax.broadcasted_iota(jnp.int32, (height, hw), 0)
            s_row = (p_row // width == h_idx).astype(jnp.float32)

            col_hw = col_ref[0:width, :]
            row_hw = row_ref[0:height, :]
            x = lax.dot_general(
                col_hw, s_col, (((0,), (0,)), ((), ())),
                preferred_element_type=jnp.float32,
            )
            y = lax.dot_general(
                row_hw, s_row, (((0,), (0,)), ((), ())),
                preferred_element_type=jnp.float32,
            )
            blk[0:num_feats, :] = x
            blk[num_feats:two_d, :] = y

        o_ref[0] = blk[...]

    return pl.pallas_call(
        body,
        grid=(batch,),
        in_specs=[
            pl.BlockSpec((table_rows, num_feats), lambda b: (0, 0)),
            pl.BlockSpec((table_rows, num_feats), lambda b: (0, 0)),
        ],
        out_specs=pl.BlockSpec((1, two_d, hw), lambda b: (b, 0, 0)),
        out_shape=jax.ShapeDtypeStruct((batch, two_d, hw), jnp.float32),
        scratch_shapes=[pltpu.VMEM((two_d, hw), jnp.float32)],
        compiler_params=pltpu.CompilerParams(
            dimension_semantics=("parallel",),
        ),
    )


def kernel(pixel_values, pixel_mask, row_embed, col_embed):
    batch = pixel_values.shape[0]
    height, width = pixel_values.shape[-2:]
    table_rows, num_feats = row_embed.shape
    call = _build_tc_call(batch, height, width, num_feats, table_rows)
    out = call(row_embed, col_embed)
    return out.reshape(batch, 2 * num_feats, height, width)
